# baseline (speedup 1.0000x reference)
import functools

import jax
import jax.numpy as jnp
from jax import lax
from jax.experimental import pallas as pl
from jax.experimental.pallas import tpu as pltpu
from jax.experimental.pallas import tpu_sc as plsc

D = 64
NC, NS = 2, 16
NW = NC * NS
BH = 256
DCH = 8


def _gather_t(idx_t, table):
    T, B = idx_t.shape
    b_per_w = B // NW
    n_pairs = T // 2

    mesh = plsc.VectorSubcoreMesh(core_axis_name="c", subcore_axis_name="s")

    @functools.partial(
        pl.kernel,
        mesh=mesh,
        out_type=jax.ShapeDtypeStruct((T, D, B), jnp.float32),
        scratch_types=[
            pltpu.VMEM((2, b_per_w), jnp.int32),
            pltpu.VMEM((4, BH, D), jnp.float32),
            pltpu.VMEM((2, D, BH), jnp.float32),
            pltpu.SemaphoreType.DMA,
            pltpu.SemaphoreType.DMA,
            pltpu.SemaphoreType.DMA,
            pltpu.SemaphoreType.DMA,
            pltpu.SemaphoreType.DMA,
            pltpu.SemaphoreType.DMA,
            pltpu.SemaphoreType.DMA,
            pltpu.SemaphoreType.DMA,
        ],
        compiler_params=pltpu.CompilerParams(
            use_tc_tiling_on_sc=False, needs_layout_passes=False),
    )
    def k(idx_hbm, table_hbm, out_hbm, idx_v, gath_v, trans_v,
          sem_i0, sem_i1, sem_g0, sem_g1, sem_g2, sem_g3, sem_s0, sem_s1):
        wid = lax.axis_index("s") * NC + lax.axis_index("c")
        b_base = wid * b_per_w
        lane = lax.iota(jnp.int32, 16)
        wrap = [(lane + dr) & 15 for dr in range(16)]
        sem_g = (sem_g0, sem_g1, sem_g2, sem_g3)
        sem_s = (sem_s0, sem_s1)
        sem_i = (sem_i0, sem_i1)

        def fire_idx(tp, t):
            pltpu.async_copy(idx_hbm.at[t, pl.ds(b_base, b_per_w)],
                             idx_v.at[tp], sem_i[tp])

        def wait_idx(tp):
            pltpu.make_async_copy(idx_hbm.at[0, pl.ds(b_base, b_per_w)],
                                  idx_v.at[tp], sem_i[tp]).wait()

        def fire_gather(tp, h, gb):
            for g in range(BH // 128):
                pltpu.async_copy(
                    table_hbm.at[idx_v.at[tp, pl.ds(h * BH + g * 128, 128)]],
                    gath_v.at[gb, pl.ds(g * 128, 128)],
                    sem_g[gb],
                )

        def wait_gather(gb):
            pltpu.make_async_copy(
                table_hbm.at[pl.ds(0, BH)], gath_v.at[gb], sem_g[gb]).wait()

        def wait_store(h):
            pltpu.make_async_copy(
                trans_v.at[h], out_hbm.at[0, :, pl.ds(b_base, BH)],
                sem_s[h]).wait()

        def transpose(gb, h):
            def tr(d0c, _):
                d0 = d0c * 16
                cols = [wrap[dr] + d0 for dr in range(16)]
                for blk in range(BH // 16):
                    rows_b = lane + blk * 16
                    vals = [plsc.load_gather(gath_v.at[gb], [rows_b, cols[dr]])
                            for dr in range(16)]
                    for dr in range(16):
                        plsc.store_scatter(trans_v.at[h],
                                           [cols[dr], rows_b], vals[dr])
                return 0
            lax.fori_loop(0, D // 16, tr, 0)

        def fire_store(t, h):
            pltpu.async_copy(
                trans_v.at[h],
                out_hbm.at[t, :, pl.ds(b_base + h * BH, BH)],
                sem_s[h],
            )

        def pair(j, _):
            t0 = 2 * j
            t1 = 2 * j + 1

            @pl.when(j > 0)
            def _wi1():
                wait_idx(1)
            fire_gather(1, 0, 2)
            wait_gather(0)

            @pl.when(j > 0)
            def _w0():
                wait_store(0)
            transpose(0, 0)
            fire_store(t0, 0)

            fire_gather(1, 1, 3)
            wait_gather(1)

            @pl.when(j > 0)
            def _w1():
                wait_store(1)
            transpose(1, 1)
            fire_store(t0, 1)

            @pl.when(j < n_pairs - 1)
            def _pf_i0():
                fire_idx(0, t0 + 2)

            @pl.when(j < n_pairs - 1)
            def _pf_g0():
                wait_idx(0)
                fire_gather(0, 0, 0)
            wait_gather(2)
            wait_store(0)
            transpose(2, 0)
            fire_store(t1, 0)

            @pl.when(j < n_pairs - 1)
            def _pf_g1():
                fire_gather(0, 1, 1)
            wait_gather(3)
            wait_store(1)
            transpose(3, 1)
            fire_store(t1, 1)

            @pl.when(j < n_pairs - 1)
            def _pf_i1():
                fire_idx(1, t1 + 2)
            return 0

        pltpu.sync_copy(idx_hbm.at[0, pl.ds(b_base, b_per_w)], idx_v.at[0])
        pltpu.sync_copy(idx_hbm.at[1, pl.ds(b_base, b_per_w)], idx_v.at[1])
        fire_gather(0, 0, 0)
        fire_gather(0, 1, 1)
        lax.fori_loop(0, n_pairs, pair, 0)
        wait_store(0)
        wait_store(1)

    return k(idx_t, table)


def kernel(inputs, time_encodings):
    idx_t = jnp.transpose(inputs.astype(jnp.int32))
    out_t = _gather_t(idx_t, time_encodings)
    return jnp.transpose(out_t, (2, 0, 1))


# speedup vs baseline: 1.0549x; 1.0549x over previous
import functools

import jax
import jax.numpy as jnp
from jax import lax
from jax.experimental import pallas as pl
from jax.experimental.pallas import tpu as pltpu
from jax.experimental.pallas import tpu_sc as plsc

D = 64
NC, NS = 2, 16
NW = NC * NS
BH = 256
DCH = 8


def _gather_t(idx_t, table):
    T, B = idx_t.shape
    b_per_w = B // NW
    n_pairs = T // 2

    mesh = plsc.VectorSubcoreMesh(core_axis_name="c", subcore_axis_name="s")

    @functools.partial(
        pl.kernel,
        mesh=mesh,
        out_type=jax.ShapeDtypeStruct((T, D, B), jnp.float32),
        scratch_types=[
            pltpu.VMEM((2, b_per_w), jnp.int32),
            pltpu.VMEM((2, BH, D), jnp.float32),
            pltpu.VMEM((2, D, BH), jnp.float32),
            pltpu.SemaphoreType.DMA,
            pltpu.SemaphoreType.DMA,
            pltpu.SemaphoreType.DMA,
            pltpu.SemaphoreType.DMA,
            pltpu.SemaphoreType.DMA,
            pltpu.SemaphoreType.DMA,
        ],
        compiler_params=pltpu.CompilerParams(
            use_tc_tiling_on_sc=False, needs_layout_passes=False),
    )
    def k(idx_hbm, table_hbm, out_hbm, idx_v, gath_v, trans_v,
          sem_i0, sem_i1, sem_g0, sem_g1, sem_s0, sem_s1):
        wid = lax.axis_index("s") * NC + lax.axis_index("c")
        b_base = wid * b_per_w
        lane = lax.iota(jnp.int32, 16)
        wrap = [(lane + dr) & 15 for dr in range(16)]
        sem_g = (sem_g0, sem_g1)
        sem_s = (sem_s0, sem_s1)
        sem_i = (sem_i0, sem_i1)

        def fire_idx(tp, t):
            pltpu.async_copy(idx_hbm.at[t, pl.ds(b_base, b_per_w)],
                             idx_v.at[tp], sem_i[tp])

        def wait_idx(tp):
            pltpu.make_async_copy(idx_hbm.at[0, pl.ds(b_base, b_per_w)],
                                  idx_v.at[tp], sem_i[tp]).wait()

        def fire_gather(tp, h):
            for g in range(BH // 128):
                pltpu.async_copy(
                    table_hbm.at[idx_v.at[tp, pl.ds(h * BH + g * 128, 128)]],
                    gath_v.at[h, pl.ds(g * 128, 128)],
                    sem_g[h],
                )

        def wait_gather(h):
            pltpu.make_async_copy(
                table_hbm.at[pl.ds(0, BH)], gath_v.at[h], sem_g[h]).wait()

        def wait_store(h):
            pltpu.make_async_copy(
                trans_v.at[h], out_hbm.at[0, :, pl.ds(b_base, BH)],
                sem_s[h]).wait()

        def transpose(h):
            def tr(d0c, _):
                d0 = d0c * 16
                cols = [wrap[dr] + d0 for dr in range(16)]
                for blk in range(BH // 16):
                    rows_b = lane + blk * 16
                    vals = [plsc.load_gather(gath_v.at[h], [rows_b, cols[dr]])
                            for dr in range(16)]
                    for dr in range(16):
                        plsc.store_scatter(trans_v.at[h],
                                           [cols[dr], rows_b], vals[dr])
                return 0
            lax.fori_loop(0, D // 16, tr, 0)

        def fire_store(t, h):
            pltpu.async_copy(
                trans_v.at[h],
                out_hbm.at[t, :, pl.ds(b_base + h * BH, BH)],
                sem_s[h],
            )

        def pair(j, _):
            t0 = 2 * j
            t1 = 2 * j + 1

            fire_idx(1, t1)
            fire_gather(0, 1)
            wait_gather(0)

            @pl.when(j > 0)
            def _w0():
                wait_store(0)
            transpose(0)
            fire_store(t0, 0)

            wait_idx(1)
            fire_gather(1, 0)
            wait_gather(1)

            @pl.when(j > 0)
            def _w1():
                wait_store(1)
            transpose(1)
            fire_store(t0, 1)

            @pl.when(j < n_pairs - 1)
            def _pf_idx():
                fire_idx(0, t1 + 1)
            fire_gather(1, 1)
            wait_gather(0)
            wait_store(0)
            transpose(0)
            fire_store(t1, 0)

            @pl.when(j < n_pairs - 1)
            def _pf_g():
                wait_idx(0)
                fire_gather(0, 0)
            wait_gather(1)
            wait_store(1)
            transpose(1)
            fire_store(t1, 1)
            return 0

        pltpu.sync_copy(idx_hbm.at[0, pl.ds(b_base, b_per_w)], idx_v.at[0])
        fire_gather(0, 0)
        lax.fori_loop(0, n_pairs, pair, 0)
        wait_store(0)
        wait_store(1)

    return k(idx_t, table)


def kernel(inputs, time_encodings):
    idx_t = jnp.transpose(inputs.astype(jnp.int32))
    out_t = _gather_t(idx_t, time_encodings)
    return jnp.transpose(out_t, (2, 0, 1))
